# baseline (speedup 1.0000x reference)
import jax
import jax.numpy as jnp
import numpy as np
from jax.experimental import pallas as pl

_SA00 = float(np.float32(0.003))
_SA01 = float(np.float32(0.001))
_SA10 = float(np.float32(0.002))
_SA11 = float(np.float32(3e-06))


def _body(x_ref, s_ref, o_ref):
    x = x_ref[...]
    s = s_ref[...]
    o = jnp.where(s == 1, _SA00, x)
    o = jnp.where(s == 2, _SA01, o)
    o = jnp.where(s == 3, _SA10, o)
    o = jnp.where(s == 4, _SA11, o)
    o_ref[...] = o


def kernel(input, p_state):
    shape = input.shape
    n = input.size
    cols = 256
    rows = n // cols
    x = input.reshape(rows, cols)
    s = p_state.reshape(rows, cols)
    blk = 4096
    grid = (rows // blk,)
    out = pl.pallas_call(
        _body,
        grid=grid,
        in_specs=[
            pl.BlockSpec((blk, cols), lambda i: (i, 0)),
            pl.BlockSpec((blk, cols), lambda i: (i, 0)),
        ],
        out_specs=pl.BlockSpec((blk, cols), lambda i: (i, 0)),
        out_shape=jax.ShapeDtypeStruct((rows, cols), jnp.float32),
    )(x, s)
    return out.reshape(shape)
